# baseline (speedup 1.0000x reference)
import functools

import jax
import jax.numpy as jnp
import numpy as np
from jax import lax
from jax.experimental import pallas as pl
from jax.experimental.pallas import tpu as pltpu
from jax.experimental.pallas import tpu_sc as plsc

_BATCH = 4096
_NFIELD = 30
_EMBED = 64
_TABLE = 30000
_KEPT_COLS = np.array(
    [0, 1, 2, 4, 5, 6, 7, 10, 11, 12, 13, 14, 17, 18, 21, 22, 23]
    + list(range(26, 39)),
    dtype=np.int32,
)
_OFFSETS = (np.arange(_NFIELD, dtype=np.int32) * 1000)

_NC, _NS = 2, 16
_NW = _NC * _NS
_CHUNK = 128
_NB = 4

_BN_C = float(1.0 / np.sqrt(1.0 + 1e-5))


def _make_sc_gather(ns):
    idx_w = ns * _NFIELD // _NW
    nchunk = idx_w // _CHUNK
    samp_w = ns // _NW
    half = _CHUNK // 2

    def sc_gather(idxq_hbm, idxt_hbm, emb_hbm, linw_hbm, rows_out, lin_out,
                  idx_v, idxt_v, rows_v, lin_v, linw_v,
                  g0, g1, g2, g3, w0, w1, w2, w3):
        gsems = (g0, g1, g2, g3)
        wsems = (w0, w1, w2, w3)
        wid = lax.axis_index("s") * _NC + lax.axis_index("c")
        pltpu.sync_copy(idxq_hbm.at[wid], idx_v)
        pltpu.sync_copy(idxt_hbm.at[wid], idxt_v)
        pltpu.sync_copy(linw_hbm, linw_v)

        out_base = wid * (idx_w // 2)

        def _start_gather(c, b):
            pltpu.async_copy(emb_hbm.at[idx_v.at[c, 0]], rows_v.at[b, 0],
                             gsems[b])
            pltpu.async_copy(emb_hbm.at[idx_v.at[c, 1]], rows_v.at[b, 1],
                             gsems[b])

        def _wait_gather(c, b):
            pltpu.make_async_copy(
                emb_hbm.at[idx_v.at[c, 0]], rows_v.at[b, 0],
                gsems[b]).wait()
            pltpu.make_async_copy(
                emb_hbm.at[idx_v.at[c, 1]], rows_v.at[b, 1],
                gsems[b]).wait()

        def _wb_descr(c, b):
            q0 = out_base + c * half
            k, s0 = q0 // ns, q0 % ns
            return (
                pltpu.make_async_copy(
                    rows_v.at[b, 0],
                    rows_out.at[k, pl.ds(s0, half), pl.ds(0, _EMBED)],
                    wsems[b]),
                pltpu.make_async_copy(
                    rows_v.at[b, 1],
                    rows_out.at[k, pl.ds(s0, half), pl.ds(_EMBED, _EMBED)],
                    wsems[b]),
            )

        def _start_wb(c, b):
            for d in _wb_descr(c, b):
                d.start()

        def _wait_wb(c, b):
            for d in _wb_descr(c, b):
                d.wait()

        for c in range(_NB - 1):
            _start_gather(c, c)

        def ring_body(i, carry):
            for b in range(_NB):
                c = _NB * i + b

                @pl.when(c < nchunk)
                def _():
                    _wait_gather(c, b)
                    _start_wb(c, b)

                cn = c + _NB - 1
                bn = (b + _NB - 1) % _NB

                @pl.when(cn < nchunk)
                def _():
                    @pl.when(cn >= _NB)
                    def _():
                        _wait_wb(cn - _NB, bn)
                    _start_gather(cn, bn)
            return carry

        lax.fori_loop(0, (nchunk + _NB - 1) // _NB, ring_body, 0)
        for c in range(max(nchunk - _NB, 0), nchunk):
            _wait_wb(c, c % _NB)

        for g in range(samp_w // 16):
            lin_v[pl.ds(g * 16, 16)] = jnp.zeros((16,), jnp.float32)

        def lin_field(f, carry):
            def lin_group(g, carry2):
                idxs = idxt_v[pl.ds(f * samp_w + g * 16, 16)]
                vals = plsc.load_gather(linw_v, [idxs])
                lin_v[pl.ds(g * 16, 16)] = lin_v[pl.ds(g * 16, 16)] + vals
                return carry2

            return lax.fori_loop(0, samp_w // 16, lin_group, carry)

        lax.fori_loop(0, _NFIELD, lin_field, 0)
        pltpu.sync_copy(lin_v, lin_out.at[pl.ds(wid * samp_w, samp_w)])

    return pl.kernel(
        sc_gather,
        out_type=[
            jax.ShapeDtypeStruct((_NFIELD // 2, ns, 2 * _EMBED),
                                 jnp.float32),
            jax.ShapeDtypeStruct((ns,), jnp.float32),
        ],
        mesh=plsc.VectorSubcoreMesh(
            core_axis_name="c", subcore_axis_name="s",
            num_cores=_NC, num_subcores=_NS),
        compiler_params=pltpu.CompilerParams(
            use_tc_tiling_on_sc=False, needs_layout_passes=False),
        scratch_types=[
            pltpu.VMEM((nchunk, 2, _CHUNK // 2), jnp.int32),
            pltpu.VMEM((idx_w,), jnp.int32),
            pltpu.VMEM((_NB, 2, _CHUNK // 2, _EMBED), jnp.float32),
            pltpu.VMEM((samp_w,), jnp.float32),
            pltpu.VMEM((_TABLE,), jnp.float32),
            pltpu.SemaphoreType.DMA, pltpu.SemaphoreType.DMA,
            pltpu.SemaphoreType.DMA, pltpu.SemaphoreType.DMA,
            pltpu.SemaphoreType.DMA, pltpu.SemaphoreType.DMA,
            pltpu.SemaphoreType.DMA, pltpu.SemaphoreType.DMA,
        ],
    )


def _mlp_body(h_ref, lin_ref, w1_ref, b1_ref, g1_ref, e1_ref,
              w2_ref, b2_ref, g2_ref, e2_ref,
              w3_ref, b3_ref, g3_ref, e3_ref,
              wo_ref, bo_ref, out_ref):
    h = jnp.concatenate(
        [h_ref[k] for k in range(_NFIELD // 2)], axis=1)
    z = jnp.dot(h.astype(jnp.bfloat16), w1_ref[...],
                preferred_element_type=jnp.float32)
    z = (z + b1_ref[...]) * (g1_ref[...] * _BN_C) + e1_ref[...]
    a = jnp.maximum(z, 0.0)
    z = jnp.dot(a.astype(jnp.bfloat16), w2_ref[...],
                preferred_element_type=jnp.float32)
    z = (z + b2_ref[...]) * (g2_ref[...] * _BN_C) + e2_ref[...]
    a = jnp.maximum(z, 0.0)
    z = jnp.dot(a.astype(jnp.bfloat16), w3_ref[...],
                preferred_element_type=jnp.float32)
    z = (z + b3_ref[...]) * (g3_ref[...] * _BN_C) + e3_ref[...]
    a = jnp.maximum(z, 0.0)
    o = jnp.dot(a.astype(jnp.bfloat16), wo_ref[...],
                preferred_element_type=jnp.float32)
    o = o + bo_ref[...] + lin_ref[...]
    out_ref[...] = 1.0 / (1.0 + jnp.exp(-o))


_BT = 512


def _mlp_call(h, lin2d, *weights):
    ns = h.shape[1]
    full = lambda shape: pl.BlockSpec(shape, lambda i: (0, 0))
    return pl.pallas_call(
        _mlp_body,
        grid=(ns // _BT,),
        in_specs=[
            pl.BlockSpec((_NFIELD // 2, _BT, 128), lambda i: (0, i, 0)),
            pl.BlockSpec((_BT, 1), lambda i: (i, 0)),
            full((1920, 512)), full((1, 512)), full((1, 512)), full((1, 512)),
            full((512, 256)), full((1, 256)), full((1, 256)), full((1, 256)),
            full((256, 128)), full((1, 128)), full((1, 128)), full((1, 128)),
            full((128, 1)), full((1, 1)),
        ],
        out_specs=pl.BlockSpec((_BT, 1), lambda i: (i, 0)),
        out_shape=jax.ShapeDtypeStruct((ns, 1), jnp.float32),
    )(h, lin2d, *weights)


_NSPLIT = 2
_NS_HALF = _BATCH // _NSPLIT
_sc_gather_half = _make_sc_gather(_NS_HALF)


def _half_idx(xi_h):
    ns = xi_h.shape[0]
    nchunk = ns * _NFIELD // (_NW * _CHUNK)
    ev = xi_h[:, 0::2].T.reshape(_NW, nchunk, _CHUNK // 2)
    od = xi_h[:, 1::2].T.reshape(_NW, nchunk, _CHUNK // 2)
    idxq = jnp.stack([ev, od], axis=2)
    idxt = (xi_h.reshape(_NW, ns // _NW, _NFIELD)
            .transpose(0, 2, 1).reshape(_NW, ns * _NFIELD // _NW))
    return idxq, idxt


def kernel(x, additional, linear_w, linear_b, emb,
           W1, b1, g1, be1, W2, b2, g2, be2, W3, b3, g3, be3, Wo, bo):
    del additional
    xi = (x[:, _KEPT_COLS].astype(jnp.int32)
          + jnp.asarray(_OFFSETS)[None, :])
    linw = linear_w.reshape(_TABLE)

    bf = jnp.bfloat16
    weights = (W1.astype(bf), b1.reshape(1, -1), g1.reshape(1, -1),
               be1.reshape(1, -1),
               W2.astype(bf), b2.reshape(1, -1), g2.reshape(1, -1),
               be2.reshape(1, -1),
               W3.astype(bf), b3.reshape(1, -1), g3.reshape(1, -1),
               be3.reshape(1, -1),
               Wo.astype(bf), bo.reshape(1, 1))

    outs = []
    for p in range(_NSPLIT):
        xi_h = xi[p * _NS_HALF:(p + 1) * _NS_HALF]
        idxq, idxt = _half_idx(xi_h)
        rows, lin = _sc_gather_half(idxq, idxt, emb, linw)
        lin2d = lin.reshape(_NS_HALF, 1) + linear_b[0]
        outs.append(_mlp_call(rows, lin2d, *weights).reshape(_NS_HALF))
    return jnp.concatenate(outs)


# speedup vs baseline: 1.1797x; 1.1797x over previous
import functools

import jax
import jax.numpy as jnp
import numpy as np
from jax import lax
from jax.experimental import pallas as pl
from jax.experimental.pallas import tpu as pltpu
from jax.experimental.pallas import tpu_sc as plsc

_BATCH = 4096
_NFIELD = 30
_EMBED = 64
_TABLE = 30000
_KEPT_COLS = np.array(
    [0, 1, 2, 4, 5, 6, 7, 10, 11, 12, 13, 14, 17, 18, 21, 22, 23]
    + list(range(26, 39)),
    dtype=np.int32,
)
_OFFSETS = (np.arange(_NFIELD, dtype=np.int32) * 1000)

_NC, _NS = 2, 16
_NW = _NC * _NS
_CHUNK = 128
_NB = 4

_BN_C = float(1.0 / np.sqrt(1.0 + 1e-5))


def _pad8(n):
    return (n + 7) // 8 * 8


def _make_sc_gather(ns):
    idx_w = ns * _NFIELD // _NW
    nchunk = idx_w // _CHUNK
    samp_w = ns // _NW
    half = _CHUNK // 2

    def sc_gather(idxq_hbm, idxt_hbm, emb_hbm, linw_hbm, rows_out, lin_out,
                  idx_v, idxt_v, rows_v, lin_v, linw_v,
                  g0, g1, g2, g3, w0, w1, w2, w3):
        gsems = (g0, g1, g2, g3)
        wsems = (w0, w1, w2, w3)
        wid = lax.axis_index("s") * _NC + lax.axis_index("c")
        pltpu.sync_copy(idxq_hbm.at[wid], idx_v)
        pltpu.sync_copy(idxt_hbm.at[wid], idxt_v)
        pltpu.sync_copy(linw_hbm, linw_v)

        out_base = wid * (idx_w // 2)

        def _start_gather(c, b):
            pltpu.async_copy(emb_hbm.at[idx_v.at[c, pl.ds(0, half)]],
                             rows_v.at[b, 0], gsems[b])
            pltpu.async_copy(emb_hbm.at[idx_v.at[c, pl.ds(half, half)]],
                             rows_v.at[b, 1], gsems[b])

        def _wait_gather(c, b):
            pltpu.make_async_copy(
                emb_hbm.at[idx_v.at[c, pl.ds(0, half)]], rows_v.at[b, 0],
                gsems[b]).wait()
            pltpu.make_async_copy(
                emb_hbm.at[idx_v.at[c, pl.ds(half, half)]], rows_v.at[b, 1],
                gsems[b]).wait()

        def _wb_descr(c, b):
            q0 = out_base + c * half
            k, s0 = q0 // ns, q0 % ns
            return (
                pltpu.make_async_copy(
                    rows_v.at[b, 0],
                    rows_out.at[k, pl.ds(s0, half), pl.ds(0, _EMBED)],
                    wsems[b]),
                pltpu.make_async_copy(
                    rows_v.at[b, 1],
                    rows_out.at[k, pl.ds(s0, half), pl.ds(_EMBED, _EMBED)],
                    wsems[b]),
            )

        def _start_wb(c, b):
            for d in _wb_descr(c, b):
                d.start()

        def _wait_wb(c, b):
            for d in _wb_descr(c, b):
                d.wait()

        for c in range(_NB - 1):
            _start_gather(c, c)

        def ring_body(i, carry):
            for b in range(_NB):
                c = _NB * i + b

                @pl.when(c < nchunk)
                def _():
                    _wait_gather(c, b)
                    _start_wb(c, b)

                cn = c + _NB - 1
                bn = (b + _NB - 1) % _NB

                @pl.when(cn < nchunk)
                def _():
                    @pl.when(cn >= _NB)
                    def _():
                        _wait_wb(cn - _NB, bn)
                    _start_gather(cn, bn)
            return carry

        lax.fori_loop(0, (nchunk + _NB - 1) // _NB, ring_body, 0)
        for c in range(max(nchunk - _NB, 0), nchunk):
            _wait_wb(c, c % _NB)

        for g in range(samp_w // 16):
            lin_v[pl.ds(g * 16, 16)] = jnp.zeros((16,), jnp.float32)

        def lin_field(f, carry):
            def lin_group(g, carry2):
                p = f * samp_w + g * 16
                idxs = idxt_v[p // 128, pl.ds(p % 128, 16)]
                vals = plsc.load_gather(linw_v, [idxs])
                lin_v[pl.ds(g * 16, 16)] = lin_v[pl.ds(g * 16, 16)] + vals
                return carry2

            return lax.fori_loop(0, samp_w // 16, lin_group, carry)

        lax.fori_loop(0, _NFIELD, lin_field, 0)
        pltpu.sync_copy(lin_v, lin_out.at[pl.ds(wid * samp_w, samp_w)])

    return pl.kernel(
        sc_gather,
        out_type=[
            jax.ShapeDtypeStruct((_NFIELD // 2, ns, 2 * _EMBED),
                                 jnp.float32),
            jax.ShapeDtypeStruct((ns,), jnp.float32),
        ],
        mesh=plsc.VectorSubcoreMesh(
            core_axis_name="c", subcore_axis_name="s",
            num_cores=_NC, num_subcores=_NS),
        compiler_params=pltpu.CompilerParams(
            use_tc_tiling_on_sc=False, needs_layout_passes=False),
        scratch_types=[
            pltpu.VMEM((_pad8(nchunk), _CHUNK), jnp.int32),
            pltpu.VMEM((_pad8(idx_w // 128), 128), jnp.int32),
            pltpu.VMEM((_NB, 2, _CHUNK // 2, _EMBED), jnp.float32),
            pltpu.VMEM((samp_w,), jnp.float32),
            pltpu.VMEM((_TABLE,), jnp.float32),
            pltpu.SemaphoreType.DMA, pltpu.SemaphoreType.DMA,
            pltpu.SemaphoreType.DMA, pltpu.SemaphoreType.DMA,
            pltpu.SemaphoreType.DMA, pltpu.SemaphoreType.DMA,
            pltpu.SemaphoreType.DMA, pltpu.SemaphoreType.DMA,
        ],
    )


def _mlp_body(h_ref, lin_ref, w1_ref, b1_ref, g1_ref, e1_ref,
              w2_ref, b2_ref, g2_ref, e2_ref,
              w3_ref, b3_ref, g3_ref, e3_ref,
              wo_ref, bo_ref, out_ref):
    h = jnp.concatenate(
        [h_ref[k] for k in range(_NFIELD // 2)], axis=1)
    z = jnp.dot(h.astype(jnp.bfloat16), w1_ref[...],
                preferred_element_type=jnp.float32)
    z = (z + b1_ref[...]) * (g1_ref[...] * _BN_C) + e1_ref[...]
    a = jnp.maximum(z, 0.0)
    z = jnp.dot(a.astype(jnp.bfloat16), w2_ref[...],
                preferred_element_type=jnp.float32)
    z = (z + b2_ref[...]) * (g2_ref[...] * _BN_C) + e2_ref[...]
    a = jnp.maximum(z, 0.0)
    z = jnp.dot(a.astype(jnp.bfloat16), w3_ref[...],
                preferred_element_type=jnp.float32)
    z = (z + b3_ref[...]) * (g3_ref[...] * _BN_C) + e3_ref[...]
    a = jnp.maximum(z, 0.0)
    o = jnp.dot(a.astype(jnp.bfloat16), wo_ref[...],
                preferred_element_type=jnp.float32)
    o = o + bo_ref[...] + lin_ref[...]
    out_ref[...] = 1.0 / (1.0 + jnp.exp(-o))


_BT = 512


def _mlp_call(h, lin2d, *weights):
    ns = h.shape[1]
    full = lambda shape: pl.BlockSpec(shape, lambda i: (0, 0))
    return pl.pallas_call(
        _mlp_body,
        grid=(ns // _BT,),
        in_specs=[
            pl.BlockSpec((_NFIELD // 2, _BT, 128), lambda i: (0, i, 0)),
            pl.BlockSpec((_BT, 1), lambda i: (i, 0)),
            full((1920, 512)), full((1, 512)), full((1, 512)), full((1, 512)),
            full((512, 256)), full((1, 256)), full((1, 256)), full((1, 256)),
            full((256, 128)), full((1, 128)), full((1, 128)), full((1, 128)),
            full((128, 1)), full((1, 1)),
        ],
        out_specs=pl.BlockSpec((_BT, 1), lambda i: (i, 0)),
        out_shape=jax.ShapeDtypeStruct((ns, 1), jnp.float32),
    )(h, lin2d, *weights)


_NSPLIT = 1
_NS_HALF = _BATCH // _NSPLIT
_sc_gather_half = _make_sc_gather(_NS_HALF)


def _half_idx(xi_h):
    ns = xi_h.shape[0]
    nchunk = ns * _NFIELD // (_NW * _CHUNK)
    ev = xi_h[:, 0::2].T.reshape(_NW, nchunk, _CHUNK // 2)
    od = xi_h[:, 1::2].T.reshape(_NW, nchunk, _CHUNK // 2)
    idxq = jnp.concatenate([ev, od], axis=2)
    idxq = jnp.pad(idxq, ((0, 0), (0, _pad8(nchunk) - nchunk), (0, 0)))
    nrow = ns * _NFIELD // (_NW * 128)
    idxt = (xi_h.reshape(_NW, ns // _NW, _NFIELD)
            .transpose(0, 2, 1).reshape(_NW, nrow, 128))
    idxt = jnp.pad(idxt, ((0, 0), (0, _pad8(nrow) - nrow), (0, 0)))
    return idxq, idxt


def kernel(x, additional, linear_w, linear_b, emb,
           W1, b1, g1, be1, W2, b2, g2, be2, W3, b3, g3, be3, Wo, bo):
    del additional
    xi = (x[:, _KEPT_COLS].astype(jnp.int32)
          + jnp.asarray(_OFFSETS)[None, :])
    linw = linear_w.reshape(_TABLE)

    bf = jnp.bfloat16
    weights = (W1.astype(bf), b1.reshape(1, -1), g1.reshape(1, -1),
               be1.reshape(1, -1),
               W2.astype(bf), b2.reshape(1, -1), g2.reshape(1, -1),
               be2.reshape(1, -1),
               W3.astype(bf), b3.reshape(1, -1), g3.reshape(1, -1),
               be3.reshape(1, -1),
               Wo.astype(bf), bo.reshape(1, 1))

    outs = []
    for p in range(_NSPLIT):
        xi_h = xi[p * _NS_HALF:(p + 1) * _NS_HALF]
        idxq, idxt = _half_idx(xi_h)
        rows, lin = _sc_gather_half(idxq, idxt, emb, linw)
        lin2d = lin.reshape(_NS_HALF, 1) + linear_b[0]
        outs.append(_mlp_call(rows, lin2d, *weights).reshape(_NS_HALF))
    return jnp.concatenate(outs)


# speedup vs baseline: 1.2391x; 1.0504x over previous
import functools

import jax
import jax.numpy as jnp
import numpy as np
from jax import lax
from jax.experimental import pallas as pl
from jax.experimental.pallas import tpu as pltpu
from jax.experimental.pallas import tpu_sc as plsc

_BATCH = 4096
_NFIELD = 30
_EMBED = 64
_TABLE = 30000
_KEPT_COLS = np.array(
    [0, 1, 2, 4, 5, 6, 7, 10, 11, 12, 13, 14, 17, 18, 21, 22, 23]
    + list(range(26, 39)),
    dtype=np.int32,
)
_OFFSETS = (np.arange(_NFIELD, dtype=np.int32) * 1000)

_NC, _NS = 2, 16
_NW = _NC * _NS
_CHUNK = 128
_NB = 4

_BN_C = float(1.0 / np.sqrt(1.0 + 1e-5))


def _pad8(n):
    return (n + 7) // 8 * 8


def _make_sc_gather(ns):
    idx_w = ns * _NFIELD // _NW
    nchunk = idx_w // _CHUNK
    samp_w = ns // _NW
    half = _CHUNK // 2

    def sc_gather(idxq_hbm, idxt_hbm, emb_hbm, linw_hbm, rows_out, lin_out,
                  idx_v, idxt_v, rows_v, lin_v, linw_v,
                  g0, g1, g2, g3, w0, w1, w2, w3):
        gsems = (g0, g1, g2, g3)
        wsems = (w0, w1, w2, w3)
        wid = lax.axis_index("s") * _NC + lax.axis_index("c")
        pltpu.sync_copy(idxq_hbm.at[wid], idx_v)
        pltpu.sync_copy(idxt_hbm.at[wid], idxt_v)
        pltpu.sync_copy(linw_hbm, linw_v)

        out_base = wid * (idx_w // 2)

        def _start_gather(c, b):
            pltpu.async_copy(emb_hbm.at[idx_v.at[c, pl.ds(0, half)]],
                             rows_v.at[b, 0], gsems[b])
            pltpu.async_copy(emb_hbm.at[idx_v.at[c, pl.ds(half, half)]],
                             rows_v.at[b, 1], gsems[b])

        def _wait_gather(c, b):
            pltpu.make_async_copy(
                emb_hbm.at[idx_v.at[c, pl.ds(0, half)]], rows_v.at[b, 0],
                gsems[b]).wait()
            pltpu.make_async_copy(
                emb_hbm.at[idx_v.at[c, pl.ds(half, half)]], rows_v.at[b, 1],
                gsems[b]).wait()

        def _wb_descr(c, b):
            q0 = out_base + c * half
            k, s0 = q0 // ns, q0 % ns
            return (
                pltpu.make_async_copy(
                    rows_v.at[b, 0],
                    rows_out.at[k, pl.ds(s0, half), pl.ds(0, _EMBED)],
                    wsems[b]),
                pltpu.make_async_copy(
                    rows_v.at[b, 1],
                    rows_out.at[k, pl.ds(s0, half), pl.ds(_EMBED, _EMBED)],
                    wsems[b]),
            )

        def _start_wb(c, b):
            for d in _wb_descr(c, b):
                d.start()

        def _wait_wb(c, b):
            for d in _wb_descr(c, b):
                d.wait()

        for c in range(_NB - 1):
            _start_gather(c, c)

        def ring_body(i, carry):
            for b in range(_NB):
                c = _NB * i + b

                @pl.when(c < nchunk)
                def _():
                    _wait_gather(c, b)
                    _start_wb(c, b)

                cn = c + _NB - 1
                bn = (b + _NB - 1) % _NB

                @pl.when(cn < nchunk)
                def _():
                    @pl.when(cn >= _NB)
                    def _():
                        _wait_wb(cn - _NB, bn)
                    _start_gather(cn, bn)
            return carry

        lax.fori_loop(0, (nchunk + _NB - 1) // _NB, ring_body, 0)
        for c in range(max(nchunk - _NB, 0), nchunk):
            _wait_wb(c, c % _NB)

        for g in range(samp_w // 16):
            lin_v[pl.ds(g * 16, 16)] = jnp.zeros((16,), jnp.float32)

        def lin_field(f, carry):
            def lin_group(g, carry2):
                p = f * samp_w + g * 16
                idxs = idxt_v[p // 128, pl.ds(p % 128, 16)]
                vals = plsc.load_gather(linw_v, [idxs])
                lin_v[pl.ds(g * 16, 16)] = lin_v[pl.ds(g * 16, 16)] + vals
                return carry2

            return lax.fori_loop(0, samp_w // 16, lin_group, carry)

        lax.fori_loop(0, _NFIELD, lin_field, 0)
        pltpu.sync_copy(lin_v, lin_out.at[pl.ds(wid * samp_w, samp_w)])

    return pl.kernel(
        sc_gather,
        out_type=[
            jax.ShapeDtypeStruct((_NFIELD // 2, ns, 2 * _EMBED),
                                 jnp.float32),
            jax.ShapeDtypeStruct((ns,), jnp.float32),
        ],
        mesh=plsc.VectorSubcoreMesh(
            core_axis_name="c", subcore_axis_name="s",
            num_cores=_NC, num_subcores=_NS),
        compiler_params=pltpu.CompilerParams(
            use_tc_tiling_on_sc=False, needs_layout_passes=False),
        scratch_types=[
            pltpu.VMEM((_pad8(nchunk), _CHUNK), jnp.int32),
            pltpu.VMEM((_pad8(idx_w // 128), 128), jnp.int32),
            pltpu.VMEM((_NB, 2, _CHUNK // 2, _EMBED), jnp.float32),
            pltpu.VMEM((samp_w,), jnp.float32),
            pltpu.VMEM((_TABLE,), jnp.float32),
            pltpu.SemaphoreType.DMA, pltpu.SemaphoreType.DMA,
            pltpu.SemaphoreType.DMA, pltpu.SemaphoreType.DMA,
            pltpu.SemaphoreType.DMA, pltpu.SemaphoreType.DMA,
            pltpu.SemaphoreType.DMA, pltpu.SemaphoreType.DMA,
        ],
    )


def _mlp_body(h_ref, lin_ref, w1_ref, b1_ref, g1_ref, e1_ref,
              w2_ref, b2_ref, g2_ref, e2_ref,
              w3_ref, b3_ref, g3_ref, e3_ref,
              wo_ref, bo_ref, out_ref):
    h = jnp.concatenate(
        [h_ref[k] for k in range(_NFIELD // 2)], axis=1)
    z = jnp.dot(h.astype(jnp.bfloat16), w1_ref[...],
                preferred_element_type=jnp.float32)
    z = (z + b1_ref[...]) * (g1_ref[...] * _BN_C) + e1_ref[...]
    a = jnp.maximum(z, 0.0)
    z = jnp.dot(a.astype(jnp.bfloat16), w2_ref[...],
                preferred_element_type=jnp.float32)
    z = (z + b2_ref[...]) * (g2_ref[...] * _BN_C) + e2_ref[...]
    a = jnp.maximum(z, 0.0)
    z = jnp.dot(a.astype(jnp.bfloat16), w3_ref[...],
                preferred_element_type=jnp.float32)
    z = (z + b3_ref[...]) * (g3_ref[...] * _BN_C) + e3_ref[...]
    a = jnp.maximum(z, 0.0)
    o = jnp.dot(a.astype(jnp.bfloat16), wo_ref[...],
                preferred_element_type=jnp.float32)
    o = o[:, 0] + bo_ref[0, 0] + lin_ref[...]
    out_ref[...] = 1.0 / (1.0 + jnp.exp(-o))


_BT = 512


def _mlp_call(h, lin2d, *weights):
    ns = h.shape[1]
    full = lambda shape: pl.BlockSpec(shape, lambda i: (0, 0))
    return pl.pallas_call(
        _mlp_body,
        grid=(ns // _BT,),
        in_specs=[
            pl.BlockSpec((_NFIELD // 2, _BT, 128), lambda i: (0, i, 0)),
            pl.BlockSpec((_BT,), lambda i: (i,)),
            full((1920, 512)), full((1, 512)), full((1, 512)), full((1, 512)),
            full((512, 256)), full((1, 256)), full((1, 256)), full((1, 256)),
            full((256, 128)), full((1, 128)), full((1, 128)), full((1, 128)),
            full((128, 1)), full((1, 1)),
        ],
        out_specs=pl.BlockSpec((_BT,), lambda i: (i,)),
        out_shape=jax.ShapeDtypeStruct((ns,), jnp.float32),
    )(h, lin2d, *weights)


_NSPLIT = 1
_NS_HALF = _BATCH // _NSPLIT
_sc_gather_half = _make_sc_gather(_NS_HALF)


def _half_idx(xi_h):
    ns = xi_h.shape[0]
    nchunk = ns * _NFIELD // (_NW * _CHUNK)
    ev = xi_h[:, 0::2].T.reshape(_NW, nchunk, _CHUNK // 2)
    od = xi_h[:, 1::2].T.reshape(_NW, nchunk, _CHUNK // 2)
    idxq = jnp.concatenate([ev, od], axis=2)
    idxq = jnp.pad(idxq, ((0, 0), (0, _pad8(nchunk) - nchunk), (0, 0)))
    nrow = ns * _NFIELD // (_NW * 128)
    idxt = (xi_h.reshape(_NW, ns // _NW, _NFIELD)
            .transpose(0, 2, 1).reshape(_NW, nrow, 128))
    idxt = jnp.pad(idxt, ((0, 0), (0, _pad8(nrow) - nrow), (0, 0)))
    return idxq, idxt


def kernel(x, additional, linear_w, linear_b, emb,
           W1, b1, g1, be1, W2, b2, g2, be2, W3, b3, g3, be3, Wo, bo):
    del additional
    emb = lax.optimization_barrier(emb)
    xi = (x[:, _KEPT_COLS].astype(jnp.int32)
          + jnp.asarray(_OFFSETS)[None, :])

    bf = jnp.bfloat16
    weights = (W1.astype(bf), b1.reshape(1, -1), g1.reshape(1, -1),
               be1.reshape(1, -1),
               W2.astype(bf), b2.reshape(1, -1), g2.reshape(1, -1),
               be2.reshape(1, -1),
               W3.astype(bf), b3.reshape(1, -1), g3.reshape(1, -1),
               be3.reshape(1, -1),
               Wo.astype(bf), (bo + linear_b).reshape(1, 1))

    outs = []
    for p in range(_NSPLIT):
        xi_h = xi[p * _NS_HALF:(p + 1) * _NS_HALF]
        idxq, idxt = _half_idx(xi_h)
        rows, lin = _sc_gather_half(idxq, idxt, emb,
                                    linear_w.reshape(_TABLE))
        outs.append(_mlp_call(rows, lin, *weights))
    return outs[0] if _NSPLIT == 1 else jnp.concatenate(outs)


# speedup vs baseline: 1.2517x; 1.0101x over previous
import functools

import jax
import jax.numpy as jnp
import numpy as np
from jax import lax
from jax.experimental import pallas as pl
from jax.experimental.pallas import tpu as pltpu
from jax.experimental.pallas import tpu_sc as plsc

_BATCH = 4096
_NFIELD = 30
_EMBED = 64
_TABLE = 30000
_KEPT_COLS = np.array(
    [0, 1, 2, 4, 5, 6, 7, 10, 11, 12, 13, 14, 17, 18, 21, 22, 23]
    + list(range(26, 39)),
    dtype=np.int32,
)
_OFFSETS = (np.arange(_NFIELD, dtype=np.int32) * 1000)

_NC, _NS = 2, 16
_NW = _NC * _NS
_CHUNK = 128
_NB = 4

_BN_C = float(1.0 / np.sqrt(1.0 + 1e-5))


def _pad8(n):
    return (n + 7) // 8 * 8


def _make_sc_gather(ns):
    idx_w = ns * _NFIELD // _NW
    nchunk = idx_w // _CHUNK
    samp_w = ns // _NW
    half = _CHUNK // 2

    def sc_gather(idxq_hbm, idxt_hbm, emb_hbm, linw_hbm, rows_out, lin_out,
                  idx_v, idxt_v, rows_v, lin_v, linw_v,
                  g0, g1, g2, g3, w0, w1, w2, w3):
        gsems = (g0, g1, g2, g3)
        wsems = (w0, w1, w2, w3)
        wid = lax.axis_index("s") * _NC + lax.axis_index("c")
        pltpu.sync_copy(idxq_hbm.at[wid], idx_v)
        pltpu.sync_copy(idxt_hbm.at[wid], idxt_v)
        pltpu.sync_copy(linw_hbm, linw_v)

        out_base = wid * (idx_w // 2)

        def _start_gather(c, b):
            pltpu.async_copy(emb_hbm.at[idx_v.at[c, pl.ds(0, half)]],
                             rows_v.at[b, 0], gsems[b])
            pltpu.async_copy(emb_hbm.at[idx_v.at[c, pl.ds(half, half)]],
                             rows_v.at[b, 1], gsems[b])

        def _wait_gather(c, b):
            pltpu.make_async_copy(
                emb_hbm.at[idx_v.at[c, pl.ds(0, half)]], rows_v.at[b, 0],
                gsems[b]).wait()
            pltpu.make_async_copy(
                emb_hbm.at[idx_v.at[c, pl.ds(half, half)]], rows_v.at[b, 1],
                gsems[b]).wait()

        def _wb_descr(c, b):
            q0 = out_base + c * half
            k, s0 = q0 // ns, q0 % ns
            return (
                pltpu.make_async_copy(
                    rows_v.at[b, 0],
                    rows_out.at[k, pl.ds(s0, half), pl.ds(0, _EMBED)],
                    wsems[b]),
                pltpu.make_async_copy(
                    rows_v.at[b, 1],
                    rows_out.at[k, pl.ds(s0, half), pl.ds(_EMBED, _EMBED)],
                    wsems[b]),
            )

        def _start_wb(c, b):
            for d in _wb_descr(c, b):
                d.start()

        def _wait_wb(c, b):
            for d in _wb_descr(c, b):
                d.wait()

        for c in range(_NB - 1):
            _start_gather(c, c)

        def ring_body(i, carry):
            for b in range(_NB):
                c = _NB * i + b

                @pl.when(c < nchunk)
                def _():
                    _wait_gather(c, b)
                    _start_wb(c, b)

                cn = c + _NB - 1
                bn = (b + _NB - 1) % _NB

                @pl.when(cn < nchunk)
                def _():
                    @pl.when(cn >= _NB)
                    def _():
                        _wait_wb(cn - _NB, bn)
                    _start_gather(cn, bn)
            return carry

        lax.fori_loop(0, (nchunk + _NB - 1) // _NB, ring_body, 0)
        for c in range(max(nchunk - _NB, 0), nchunk):
            _wait_wb(c, c % _NB)

        for g in range(samp_w // 16):
            lin_v[pl.ds(g * 16, 16)] = jnp.zeros((16,), jnp.float32)

        def lin_field(f, carry):
            def lin_group(g, carry2):
                p = f * samp_w + g * 16
                idxs = idxt_v[p // 128, pl.ds(p % 128, 16)]
                vals = plsc.load_gather(linw_v, [idxs])
                lin_v[pl.ds(g * 16, 16)] = lin_v[pl.ds(g * 16, 16)] + vals
                return carry2

            return lax.fori_loop(0, samp_w // 16, lin_group, carry)

        lax.fori_loop(0, _NFIELD, lin_field, 0)
        pltpu.sync_copy(lin_v, lin_out.at[pl.ds(wid * samp_w, samp_w)])

    return pl.kernel(
        sc_gather,
        out_type=[
            jax.ShapeDtypeStruct((_NFIELD // 2, ns, 2 * _EMBED),
                                 jnp.float32),
            jax.ShapeDtypeStruct((ns,), jnp.float32),
        ],
        mesh=plsc.VectorSubcoreMesh(
            core_axis_name="c", subcore_axis_name="s",
            num_cores=_NC, num_subcores=_NS),
        compiler_params=pltpu.CompilerParams(
            use_tc_tiling_on_sc=False, needs_layout_passes=False),
        scratch_types=[
            pltpu.VMEM((_pad8(nchunk), _CHUNK), jnp.int32),
            pltpu.VMEM((_pad8(idx_w // 128), 128), jnp.int32),
            pltpu.VMEM((_NB, 2, _CHUNK // 2, _EMBED), jnp.float32),
            pltpu.VMEM((samp_w,), jnp.float32),
            pltpu.VMEM((_TABLE,), jnp.float32),
            pltpu.SemaphoreType.DMA, pltpu.SemaphoreType.DMA,
            pltpu.SemaphoreType.DMA, pltpu.SemaphoreType.DMA,
            pltpu.SemaphoreType.DMA, pltpu.SemaphoreType.DMA,
            pltpu.SemaphoreType.DMA, pltpu.SemaphoreType.DMA,
        ],
    )


def _mlp_body(h_ref, lin_ref, w1_ref, b1_ref, g1_ref, e1_ref,
              w2_ref, b2_ref, g2_ref, e2_ref,
              w3_ref, b3_ref, g3_ref, e3_ref,
              wo_ref, bo_ref, out_ref):
    h = jnp.concatenate(
        [h_ref[k] for k in range(_NFIELD // 2)], axis=1)
    z = jnp.dot(h.astype(jnp.bfloat16), w1_ref[...],
                preferred_element_type=jnp.float32)
    z = (z + b1_ref[...]) * (g1_ref[...] * _BN_C) + e1_ref[...]
    a = jnp.maximum(z, 0.0)
    z = jnp.dot(a.astype(jnp.bfloat16), w2_ref[...],
                preferred_element_type=jnp.float32)
    z = (z + b2_ref[...]) * (g2_ref[...] * _BN_C) + e2_ref[...]
    a = jnp.maximum(z, 0.0)
    z = jnp.dot(a.astype(jnp.bfloat16), w3_ref[...],
                preferred_element_type=jnp.float32)
    z = (z + b3_ref[...]) * (g3_ref[...] * _BN_C) + e3_ref[...]
    a = jnp.maximum(z, 0.0)
    o = jnp.dot(a.astype(jnp.bfloat16), wo_ref[...],
                preferred_element_type=jnp.float32)
    o = o[:, 0] + bo_ref[0, 0] + lin_ref[...]
    out_ref[...] = 1.0 / (1.0 + jnp.exp(-o))


_BT = 1024


def _mlp_call(h, lin2d, *weights):
    ns = h.shape[1]
    full = lambda shape: pl.BlockSpec(shape, lambda i: (0, 0))
    return pl.pallas_call(
        _mlp_body,
        grid=(ns // _BT,),
        in_specs=[
            pl.BlockSpec((_NFIELD // 2, _BT, 128), lambda i: (0, i, 0)),
            pl.BlockSpec((_BT,), lambda i: (i,)),
            full((1920, 512)), full((1, 512)), full((1, 512)), full((1, 512)),
            full((512, 256)), full((1, 256)), full((1, 256)), full((1, 256)),
            full((256, 128)), full((1, 128)), full((1, 128)), full((1, 128)),
            full((128, 1)), full((1, 1)),
        ],
        out_specs=pl.BlockSpec((_BT,), lambda i: (i,)),
        out_shape=jax.ShapeDtypeStruct((ns,), jnp.float32),
    )(h, lin2d, *weights)


_NSPLIT = 1
_NS_HALF = _BATCH // _NSPLIT
_sc_gather_half = _make_sc_gather(_NS_HALF)


def _half_idx(xi_h):
    ns = xi_h.shape[0]
    nchunk = ns * _NFIELD // (_NW * _CHUNK)
    ev = xi_h[:, 0::2].T.reshape(_NW, nchunk, _CHUNK // 2)
    od = xi_h[:, 1::2].T.reshape(_NW, nchunk, _CHUNK // 2)
    idxq = jnp.concatenate([ev, od], axis=2)
    idxq = jnp.pad(idxq, ((0, 0), (0, _pad8(nchunk) - nchunk), (0, 0)))
    nrow = ns * _NFIELD // (_NW * 128)
    idxt = (xi_h.reshape(_NW, ns // _NW, _NFIELD)
            .transpose(0, 2, 1).reshape(_NW, nrow, 128))
    idxt = jnp.pad(idxt, ((0, 0), (0, _pad8(nrow) - nrow), (0, 0)))
    return idxq, idxt


def kernel(x, additional, linear_w, linear_b, emb,
           W1, b1, g1, be1, W2, b2, g2, be2, W3, b3, g3, be3, Wo, bo):
    del additional
    emb = lax.optimization_barrier(emb)
    xi = (x[:, _KEPT_COLS].astype(jnp.int32)
          + jnp.asarray(_OFFSETS)[None, :])

    bf = jnp.bfloat16
    weights = (W1.astype(bf), b1.reshape(1, -1), g1.reshape(1, -1),
               be1.reshape(1, -1),
               W2.astype(bf), b2.reshape(1, -1), g2.reshape(1, -1),
               be2.reshape(1, -1),
               W3.astype(bf), b3.reshape(1, -1), g3.reshape(1, -1),
               be3.reshape(1, -1),
               Wo.astype(bf), (bo + linear_b).reshape(1, 1))

    outs = []
    for p in range(_NSPLIT):
        xi_h = xi[p * _NS_HALF:(p + 1) * _NS_HALF]
        idxq, idxt = _half_idx(xi_h)
        rows, lin = _sc_gather_half(idxq, idxt, emb,
                                    linear_w.reshape(_TABLE))
        outs.append(_mlp_call(rows, lin, *weights))
    return outs[0] if _NSPLIT == 1 else jnp.concatenate(outs)
